# initial kernel scaffold (unmeasured)
import jax
import jax.numpy as jnp
from jax import lax
from jax.experimental import pallas as pl
from jax.experimental.pallas import tpu as pltpu


def kernel(
    x,
):
    def body(*refs):
        pass

    out_shape = jax.ShapeDtypeStruct(..., jnp.float32)
    return pl.pallas_call(body, out_shape=out_shape)(...)


# baseline (device time: 53889 ns/iter reference)
import jax
import jax.numpy as jnp
from jax import lax
from jax.experimental import pallas as pl
from jax.experimental.pallas import tpu as pltpu


def kernel(x):
    m, n = x.shape

    def body(x_ref, out_ref, send_sem, recv_sem):
        my_x = lax.axis_index("x")
        my_y = lax.axis_index("y")
        nbr_y = 1 - my_y

        barrier_sem = pltpu.get_barrier_semaphore()
        pl.semaphore_signal(
            barrier_sem, inc=1,
            device_id=(my_x, nbr_y), device_id_type=pl.DeviceIdType.MESH,
        )
        pl.semaphore_wait(barrier_sem, 1)

        rdma = pltpu.make_async_remote_copy(
            src_ref=x_ref,
            dst_ref=out_ref.at[pl.ds(my_y * m, m), :],
            send_sem=send_sem,
            recv_sem=recv_sem,
            device_id=(my_x, nbr_y),
            device_id_type=pl.DeviceIdType.MESH,
        )
        rdma.start()

        out_ref[pl.ds(my_y * m, m), :] = x_ref[...]

        rdma.wait()

    return pl.pallas_call(
        body,
        out_shape=jax.ShapeDtypeStruct((2 * m, n), x.dtype),
        in_specs=[pl.BlockSpec(memory_space=pltpu.VMEM)],
        out_specs=pl.BlockSpec(memory_space=pltpu.VMEM),
        scratch_shapes=[
            pltpu.SemaphoreType.DMA,
            pltpu.SemaphoreType.DMA,
        ],
        compiler_params=pltpu.CompilerParams(collective_id=0),
    )(x)


# device time: 36960 ns/iter; 1.4580x vs baseline; 1.4580x over previous
import jax
import jax.numpy as jnp
from jax import lax
from jax.experimental import pallas as pl
from jax.experimental.pallas import tpu as pltpu

C = 8


def kernel(x):
    m, n = x.shape
    m2 = m // 2
    r = m2 // C

    def body(x_ref, out_ref, p1_send, p1_recv, p2_send, p2_recv):
        my_x = lax.axis_index("x")
        my_y = lax.axis_index("y")
        nbr_y = 1 - my_y
        nbr_x = 1 - my_x
        half = my_x * m2

        barrier_sem = pltpu.get_barrier_semaphore()
        pl.semaphore_signal(
            barrier_sem, inc=1,
            device_id=(my_x, nbr_y), device_id_type=pl.DeviceIdType.MESH,
        )
        pl.semaphore_signal(
            barrier_sem, inc=1,
            device_id=(nbr_x, my_y), device_id_type=pl.DeviceIdType.MESH,
        )
        pl.semaphore_wait(barrier_sem, 2)

        p1 = []
        for c in range(C):
            rdma = pltpu.make_async_remote_copy(
                src_ref=x_ref.at[pl.ds(half + c * r, r), :],
                dst_ref=out_ref.at[pl.ds(my_y * m + half + c * r, r), :],
                send_sem=p1_send.at[c],
                recv_sem=p1_recv.at[c],
                device_id=(my_x, nbr_y),
                device_id_type=pl.DeviceIdType.MESH,
            )
            rdma.start()
            p1.append(rdma)

        out_ref[pl.ds(my_y * m, m), :] = x_ref[...]

        p2 = []
        for c in range(C):
            inb = out_ref.at[pl.ds(nbr_y * m + half + c * r, r), :]
            recv = pltpu.make_async_remote_copy(
                src_ref=x_ref.at[pl.ds(c * r, r), :],
                dst_ref=inb,
                send_sem=p1_send.at[c],
                recv_sem=p1_recv.at[c],
                device_id=(my_x, nbr_y),
                device_id_type=pl.DeviceIdType.MESH,
            )
            recv.wait_recv()
            fwd = pltpu.make_async_remote_copy(
                src_ref=inb,
                dst_ref=inb,
                send_sem=p2_send.at[c],
                recv_sem=p2_recv.at[c],
                device_id=(nbr_x, my_y),
                device_id_type=pl.DeviceIdType.MESH,
            )
            fwd.start()
            p2.append(fwd)

        for c in range(C):
            inb2 = out_ref.at[pl.ds(nbr_y * m + nbr_x * m2 + c * r, r), :]
            recv2 = pltpu.make_async_remote_copy(
                src_ref=x_ref.at[pl.ds(c * r, r), :],
                dst_ref=inb2,
                send_sem=p2_send.at[c],
                recv_sem=p2_recv.at[c],
                device_id=(nbr_x, my_y),
                device_id_type=pl.DeviceIdType.MESH,
            )
            recv2.wait_recv()
        for c in range(C):
            p1[c].wait_send()
            p2[c].wait_send()

    return pl.pallas_call(
        body,
        out_shape=jax.ShapeDtypeStruct((2 * m, n), x.dtype),
        in_specs=[pl.BlockSpec(memory_space=pltpu.VMEM)],
        out_specs=pl.BlockSpec(memory_space=pltpu.VMEM),
        scratch_shapes=[
            pltpu.SemaphoreType.DMA((C,)),
            pltpu.SemaphoreType.DMA((C,)),
            pltpu.SemaphoreType.DMA((C,)),
            pltpu.SemaphoreType.DMA((C,)),
        ],
        compiler_params=pltpu.CompilerParams(collective_id=0),
    )(x)


# device time: 35871 ns/iter; 1.5023x vs baseline; 1.0304x over previous
import jax
import jax.numpy as jnp
from jax import lax
from jax.experimental import pallas as pl
from jax.experimental.pallas import tpu as pltpu

C = 16


def kernel(x):
    m, n = x.shape
    m2 = m // 2
    r = m2 // C

    def body(x_ref, out_ref, p1_send, p1_recv, p2_send, p2_recv, loc_sem):
        my_x = lax.axis_index("x")
        my_y = lax.axis_index("y")
        nbr_y = 1 - my_y
        nbr_x = 1 - my_x
        half = my_x * m2

        barrier_sem = pltpu.get_barrier_semaphore()
        pl.semaphore_signal(
            barrier_sem, inc=1,
            device_id=(my_x, nbr_y), device_id_type=pl.DeviceIdType.MESH,
        )
        pl.semaphore_signal(
            barrier_sem, inc=1,
            device_id=(nbr_x, my_y), device_id_type=pl.DeviceIdType.MESH,
        )
        pl.semaphore_wait(barrier_sem, 2)

        p1 = []
        for c in range(C):
            rdma = pltpu.make_async_remote_copy(
                src_ref=x_ref.at[pl.ds(half + c * r, r), :],
                dst_ref=out_ref.at[pl.ds(my_y * m + half + c * r, r), :],
                send_sem=p1_send.at[c],
                recv_sem=p1_recv.at[c],
                device_id=(my_x, nbr_y),
                device_id_type=pl.DeviceIdType.MESH,
            )
            rdma.start()
            p1.append(rdma)

        loc = pltpu.make_async_copy(
            x_ref, out_ref.at[pl.ds(my_y * m, m), :], loc_sem
        )
        loc.start()

        p2 = []
        for c in range(C):
            inb = out_ref.at[pl.ds(nbr_y * m + half + c * r, r), :]
            recv = pltpu.make_async_remote_copy(
                src_ref=x_ref.at[pl.ds(c * r, r), :],
                dst_ref=inb,
                send_sem=p1_send.at[c],
                recv_sem=p1_recv.at[c],
                device_id=(my_x, nbr_y),
                device_id_type=pl.DeviceIdType.MESH,
            )
            recv.wait_recv()
            fwd = pltpu.make_async_remote_copy(
                src_ref=inb,
                dst_ref=inb,
                send_sem=p2_send.at[c],
                recv_sem=p2_recv.at[c],
                device_id=(nbr_x, my_y),
                device_id_type=pl.DeviceIdType.MESH,
            )
            fwd.start()
            p2.append(fwd)

        for c in range(C):
            inb2 = out_ref.at[pl.ds(nbr_y * m + nbr_x * m2 + c * r, r), :]
            recv2 = pltpu.make_async_remote_copy(
                src_ref=x_ref.at[pl.ds(c * r, r), :],
                dst_ref=inb2,
                send_sem=p2_send.at[c],
                recv_sem=p2_recv.at[c],
                device_id=(nbr_x, my_y),
                device_id_type=pl.DeviceIdType.MESH,
            )
            recv2.wait_recv()
        for c in range(C):
            p1[c].wait_send()
            p2[c].wait_send()
        loc.wait()

    return pl.pallas_call(
        body,
        out_shape=jax.ShapeDtypeStruct((2 * m, n), x.dtype),
        in_specs=[pl.BlockSpec(memory_space=pltpu.VMEM)],
        out_specs=pl.BlockSpec(memory_space=pltpu.VMEM),
        scratch_shapes=[
            pltpu.SemaphoreType.DMA((C,)),
            pltpu.SemaphoreType.DMA((C,)),
            pltpu.SemaphoreType.DMA((C,)),
            pltpu.SemaphoreType.DMA((C,)),
            pltpu.SemaphoreType.DMA,
        ],
        compiler_params=pltpu.CompilerParams(collective_id=0),
    )(x)
